# baseline (speedup 1.0000x reference)
import jax
import jax.numpy as jnp
from jax import lax
from jax.experimental import pallas as pl
from jax.experimental.pallas import tpu as pltpu
from jax.experimental.pallas import tpu_sc as plsc

MAX_LEN = 4096
D_MODEL = 1024
BATCH = 4

NUM_CORES = 2
NUM_SUBCORES = 16
NUM_WORKERS = NUM_CORES * NUM_SUBCORES
ROWS_PER_WORKER = MAX_LEN // NUM_WORKERS
CHUNK = 64
NUM_CHUNKS = ROWS_PER_WORKER // CHUNK


def _sc_broadcast(table_hbm, out_hbm, buf, sem):
    wid = lax.axis_index("s") * NUM_CORES + lax.axis_index("c")
    base = wid * ROWS_PER_WORKER
    for c in range(NUM_CHUNKS):
        row = base + c * CHUNK
        pltpu.sync_copy(table_hbm.at[pl.ds(row, CHUNK)], buf)
        copies = [
            pltpu.async_copy(buf, out_hbm.at[b, pl.ds(row, CHUNK)], sem)
            for b in range(BATCH)
        ]
        for cp in copies:
            cp.wait()


def kernel(x, emb_weight):
    del x
    f = pl.kernel(
        _sc_broadcast,
        out_type=jax.ShapeDtypeStruct((BATCH, MAX_LEN, D_MODEL), jnp.float32),
        mesh=plsc.VectorSubcoreMesh(core_axis_name="c", subcore_axis_name="s"),
        scratch_types=[
            pltpu.VMEM((CHUNK, D_MODEL), jnp.float32),
            pltpu.SemaphoreType.DMA,
        ],
    )
    return f(emb_weight)
